# Initial kernel scaffold; baseline (speedup 1.0000x reference)
import jax
import jax.numpy as jnp
from jax.experimental import pallas as pl


def kernel(input_molecule_representations, graph_representations, graphs_requiring_node_choices, W1, b1, W2, b2):
    raise NotImplementedError("write your pallas kernel here")


# speedup vs baseline: 1.3233x; 1.3233x over previous
import functools

import jax
import jax.numpy as jnp
from jax import lax
from jax.experimental import pallas as pl
from jax.experimental.pallas import tpu as pltpu
from jax.experimental.pallas import tpu_sc as plsc

D = 256
HID = 256
OUT = 64

NW = 32
CHUNK = 112
CHUNKS_PER_W = 7
ROWS_PER_W = CHUNK * CHUNKS_PER_W
N_PAD = NW * ROWS_PER_W

TC_BLOCK = 512


def _sc_gather(imr_hbm, gr_hbm, idx_hbm, x1_hbm, x2_hbm,
               idx_v, buf1, buf2, sem):
    wid = lax.axis_index("s") * 2 + lax.axis_index("c")
    base = wid * ROWS_PER_W
    pltpu.sync_copy(idx_hbm.at[pl.ds(base, ROWS_PER_W)], idx_v)
    for c in range(CHUNKS_PER_W):
        row0 = base + c * CHUNK
        idx_c = idx_v.at[pl.ds(c * CHUNK, CHUNK)]
        d1 = pltpu.async_copy(imr_hbm.at[idx_c], buf1, sem)
        d2 = pltpu.async_copy(gr_hbm.at[idx_c], buf2, sem)
        d1.wait()
        d2.wait()
        pltpu.sync_copy(buf1, x1_hbm.at[pl.ds(row0, CHUNK)])
        pltpu.sync_copy(buf2, x2_hbm.at[pl.ds(row0, CHUNK)])


@functools.partial(jax.jit, static_argnums=())
def _gather_rows(imr, gr, idx2d):
    mesh = plsc.VectorSubcoreMesh(core_axis_name="c", subcore_axis_name="s")
    f = pl.kernel(
        _sc_gather,
        out_type=[
            jax.ShapeDtypeStruct((N_PAD, D), jnp.float32),
            jax.ShapeDtypeStruct((N_PAD, D), jnp.float32),
        ],
        mesh=mesh,
        scratch_types=[
            pltpu.VMEM((ROWS_PER_W,), jnp.int32),
            pltpu.VMEM((CHUNK, D), jnp.float32),
            pltpu.VMEM((CHUNK, D), jnp.float32),
            pltpu.SemaphoreType.DMA,
        ],
    )
    return f(imr, gr, idx2d)


def _mlp_body(x1_ref, x2_ref, w1a_ref, w1b_ref, w2_ref, b1_ref, b2_ref, o_ref):
    h = jnp.dot(x1_ref[...], w1a_ref[...], preferred_element_type=jnp.float32)
    h += jnp.dot(x2_ref[...], w1b_ref[...], preferred_element_type=jnp.float32)
    h = jnp.maximum(h + b1_ref[...], 0.0)
    o_ref[...] = (
        jnp.dot(h, w2_ref[...], preferred_element_type=jnp.float32)
        + b2_ref[...]
    )


def _mlp(x1, x2, w1a, w1b, w2, b1r, b2r):
    n = x1.shape[0]
    grid = (n // TC_BLOCK,)
    return pl.pallas_call(
        _mlp_body,
        grid=grid,
        in_specs=[
            pl.BlockSpec((TC_BLOCK, D), lambda i: (i, 0)),
            pl.BlockSpec((TC_BLOCK, D), lambda i: (i, 0)),
            pl.BlockSpec((D, HID), lambda i: (0, 0)),
            pl.BlockSpec((D, HID), lambda i: (0, 0)),
            pl.BlockSpec((HID, OUT), lambda i: (0, 0)),
            pl.BlockSpec((1, HID), lambda i: (0, 0)),
            pl.BlockSpec((1, OUT), lambda i: (0, 0)),
        ],
        out_specs=pl.BlockSpec((TC_BLOCK, OUT), lambda i: (i, 0)),
        out_shape=jax.ShapeDtypeStruct((n, OUT), jnp.float32),
    )(x1, x2, w1a, w1b, w2, b1r, b2r)


def kernel(input_molecule_representations, graph_representations,
           graphs_requiring_node_choices, W1, b1, W2, b2):
    n_sel = graphs_requiring_node_choices.shape[0]
    idx = graphs_requiring_node_choices.astype(jnp.int32)
    idx_pad = jnp.concatenate(
        [idx, jnp.zeros((N_PAD - n_sel,), jnp.int32)])
    x1, x2 = _gather_rows(
        input_molecule_representations, graph_representations, idx_pad)
    w1a = W1[:D]
    w1b = W1[D:]
    out = _mlp(x1, x2, w1a, w1b, W2,
               b1.reshape(1, HID), b2.reshape(1, OUT))
    return out[:n_sel]


# speedup vs baseline: 1.3330x; 1.0073x over previous
import functools

import jax
import jax.numpy as jnp
from jax import lax
from jax.experimental import pallas as pl
from jax.experimental.pallas import tpu as pltpu
from jax.experimental.pallas import tpu_sc as plsc

D = 256
HID = 256
OUT = 64

NW = 32
CHUNK = 112
CHUNKS_PER_W = 7
ROWS_PER_W = CHUNK * CHUNKS_PER_W
N_PAD = NW * ROWS_PER_W

TC_BLOCK = 512


def _sc_gather(imr_hbm, gr_hbm, idx_hbm, x1_hbm, x2_hbm,
               idx_v, b1a, b1b, b2a, b2b, sem_g, sem_wa, sem_wb):
    wid = lax.axis_index("s") * 2 + lax.axis_index("c")
    base = wid * ROWS_PER_W
    bufs1 = (b1a, b1b)
    bufs2 = (b2a, b2b)
    sem_w = (sem_wa, sem_wb)
    pltpu.sync_copy(idx_hbm.at[pl.ds(base, ROWS_PER_W)], idx_v)
    idx0 = idx_v.at[pl.ds(0, CHUNK)]
    gcur = [pltpu.async_copy(imr_hbm.at[idx0], bufs1[0], sem_g),
            pltpu.async_copy(gr_hbm.at[idx0], bufs2[0], sem_g)]
    pending = []
    for c in range(CHUNKS_PER_W):
        cur = c % 2
        nxt = (c + 1) % 2
        for d in gcur:
            d.wait()
        row0 = base + c * CHUNK
        pending.append([
            pltpu.async_copy(bufs1[cur], x1_hbm.at[pl.ds(row0, CHUNK)],
                             sem_w[cur]),
            pltpu.async_copy(bufs2[cur], x2_hbm.at[pl.ds(row0, CHUNK)],
                             sem_w[cur]),
        ])
        if c + 1 < CHUNKS_PER_W:
            if len(pending) >= 2:
                for d in pending.pop(0):
                    d.wait()
            idx_c = idx_v.at[pl.ds((c + 1) * CHUNK, CHUNK)]
            gcur = [pltpu.async_copy(imr_hbm.at[idx_c], bufs1[nxt], sem_g),
                    pltpu.async_copy(gr_hbm.at[idx_c], bufs2[nxt], sem_g)]
    for grp in pending:
        for d in grp:
            d.wait()


@functools.partial(jax.jit, static_argnums=())
def _gather_rows(imr, gr, idx2d):
    mesh = plsc.VectorSubcoreMesh(core_axis_name="c", subcore_axis_name="s")
    f = pl.kernel(
        _sc_gather,
        out_type=[
            jax.ShapeDtypeStruct((N_PAD, D), jnp.float32),
            jax.ShapeDtypeStruct((N_PAD, D), jnp.float32),
        ],
        mesh=mesh,
        scratch_types=[
            pltpu.VMEM((ROWS_PER_W,), jnp.int32),
            pltpu.VMEM((CHUNK, D), jnp.float32),
            pltpu.VMEM((CHUNK, D), jnp.float32),
            pltpu.VMEM((CHUNK, D), jnp.float32),
            pltpu.VMEM((CHUNK, D), jnp.float32),
            pltpu.SemaphoreType.DMA,
            pltpu.SemaphoreType.DMA,
            pltpu.SemaphoreType.DMA,
        ],
        compiler_params=pltpu.CompilerParams(use_tc_tiling_on_sc=True),
    )
    return f(imr, gr, idx2d)


def _mlp_body(x1_ref, x2_ref, w1a_ref, w1b_ref, w2_ref, b1_ref, b2_ref, o_ref):
    h = jnp.dot(x1_ref[...], w1a_ref[...], preferred_element_type=jnp.float32)
    h += jnp.dot(x2_ref[...], w1b_ref[...], preferred_element_type=jnp.float32)
    h = jnp.maximum(h + b1_ref[...], 0.0)
    o_ref[...] = (
        jnp.dot(h, w2_ref[...], preferred_element_type=jnp.float32)
        + b2_ref[...]
    )


def _mlp(x1, x2, w1a, w1b, w2, b1r, b2r):
    n = x1.shape[0]
    grid = (n // TC_BLOCK,)
    return pl.pallas_call(
        _mlp_body,
        grid=grid,
        in_specs=[
            pl.BlockSpec((TC_BLOCK, D), lambda i: (i, 0)),
            pl.BlockSpec((TC_BLOCK, D), lambda i: (i, 0)),
            pl.BlockSpec((D, HID), lambda i: (0, 0)),
            pl.BlockSpec((D, HID), lambda i: (0, 0)),
            pl.BlockSpec((HID, OUT), lambda i: (0, 0)),
            pl.BlockSpec((1, HID), lambda i: (0, 0)),
            pl.BlockSpec((1, OUT), lambda i: (0, 0)),
        ],
        out_specs=pl.BlockSpec((TC_BLOCK, OUT), lambda i: (i, 0)),
        out_shape=jax.ShapeDtypeStruct((n, OUT), jnp.float32),
    )(x1, x2, w1a, w1b, w2, b1r, b2r)


def kernel(input_molecule_representations, graph_representations,
           graphs_requiring_node_choices, W1, b1, W2, b2):
    n_sel = graphs_requiring_node_choices.shape[0]
    idx = graphs_requiring_node_choices.astype(jnp.int32)
    idx_pad = jnp.concatenate(
        [idx, jnp.zeros((N_PAD - n_sel,), jnp.int32)])
    x1, x2 = _gather_rows(
        input_molecule_representations, graph_representations, idx_pad)
    w1a = W1[:D]
    w1b = W1[D:]
    out = _mlp(x1, x2, w1a, w1b, W2,
               b1.reshape(1, HID), b2.reshape(1, OUT))
    return out[:n_sel]


# speedup vs baseline: 1.5974x; 1.1983x over previous
import functools

import jax
import jax.numpy as jnp
from jax import lax
from jax.experimental import pallas as pl
from jax.experimental.pallas import tpu as pltpu
from jax.experimental.pallas import tpu_sc as plsc

D = 256
HID = 256
OUT = 64

NW = 32
CHUNK = 112
CHUNKS_PER_W = 7
ROWS_PER_W = CHUNK * CHUNKS_PER_W
N_PAD = NW * ROWS_PER_W

TC_BLOCK = 1000


def _sc_gather(imr_hbm, gr_hbm, idx_hbm, x1_hbm, x2_hbm,
               idx_v, b1a, b1b, b2a, b2b, sem_g, sem_wa, sem_wb):
    wid = lax.axis_index("s") * 2 + lax.axis_index("c")
    base = wid * ROWS_PER_W
    bufs1 = (b1a, b1b)
    bufs2 = (b2a, b2b)
    sem_w = (sem_wa, sem_wb)
    pltpu.sync_copy(idx_hbm.at[pl.ds(base, ROWS_PER_W)], idx_v)
    idx0 = idx_v.at[pl.ds(0, CHUNK)]
    gcur = [pltpu.async_copy(imr_hbm.at[idx0], bufs1[0], sem_g),
            pltpu.async_copy(gr_hbm.at[idx0], bufs2[0], sem_g)]
    pending = []
    for c in range(CHUNKS_PER_W):
        cur = c % 2
        nxt = (c + 1) % 2
        for d in gcur:
            d.wait()
        row0 = base + c * CHUNK
        pending.append([
            pltpu.async_copy(bufs1[cur], x1_hbm.at[pl.ds(row0, CHUNK)],
                             sem_w[cur]),
            pltpu.async_copy(bufs2[cur], x2_hbm.at[pl.ds(row0, CHUNK)],
                             sem_w[cur]),
        ])
        if c + 1 < CHUNKS_PER_W:
            if len(pending) >= 2:
                for d in pending.pop(0):
                    d.wait()
            idx_c = idx_v.at[pl.ds((c + 1) * CHUNK, CHUNK)]
            gcur = [pltpu.async_copy(imr_hbm.at[idx_c], bufs1[nxt], sem_g),
                    pltpu.async_copy(gr_hbm.at[idx_c], bufs2[nxt], sem_g)]
    for grp in pending:
        for d in grp:
            d.wait()


@functools.partial(jax.jit, static_argnums=())
def _gather_rows(imr, gr, idx2d):
    mesh = plsc.VectorSubcoreMesh(core_axis_name="c", subcore_axis_name="s")
    f = pl.kernel(
        _sc_gather,
        out_type=[
            jax.ShapeDtypeStruct((N_PAD, D), jnp.float32),
            jax.ShapeDtypeStruct((N_PAD, D), jnp.float32),
        ],
        mesh=mesh,
        scratch_types=[
            pltpu.VMEM((ROWS_PER_W,), jnp.int32),
            pltpu.VMEM((CHUNK, D), jnp.float32),
            pltpu.VMEM((CHUNK, D), jnp.float32),
            pltpu.VMEM((CHUNK, D), jnp.float32),
            pltpu.VMEM((CHUNK, D), jnp.float32),
            pltpu.SemaphoreType.DMA,
            pltpu.SemaphoreType.DMA,
            pltpu.SemaphoreType.DMA,
        ],
        compiler_params=pltpu.CompilerParams(use_tc_tiling_on_sc=True),
    )
    return f(imr, gr, idx2d)


def _mlp_body(x1_ref, x2_ref, w1a_ref, w1b_ref, w2_ref, b1_ref, b2_ref, o_ref):
    h = jnp.dot(x1_ref[...], w1a_ref[...], preferred_element_type=jnp.float32)
    h += jnp.dot(x2_ref[...], w1b_ref[...], preferred_element_type=jnp.float32)
    h = jnp.maximum(h + b1_ref[...], 0.0)
    o_ref[...] = (
        jnp.dot(h, w2_ref[...], preferred_element_type=jnp.float32)
        + b2_ref[...]
    )


def _mlp(x1, x2, w1a, w1b, w2, b1r, b2r, n):
    grid = (n // TC_BLOCK,)
    return pl.pallas_call(
        _mlp_body,
        grid=grid,
        in_specs=[
            pl.BlockSpec((TC_BLOCK, D), lambda i: (i, 0)),
            pl.BlockSpec((TC_BLOCK, D), lambda i: (i, 0)),
            pl.BlockSpec((D, HID), lambda i: (0, 0)),
            pl.BlockSpec((D, HID), lambda i: (0, 0)),
            pl.BlockSpec((HID, OUT), lambda i: (0, 0)),
            pl.BlockSpec((1, HID), lambda i: (0, 0)),
            pl.BlockSpec((1, OUT), lambda i: (0, 0)),
        ],
        out_specs=pl.BlockSpec((TC_BLOCK, OUT), lambda i: (i, 0)),
        out_shape=jax.ShapeDtypeStruct((n, OUT), jnp.float32),
    )(x1, x2, w1a, w1b, w2, b1r, b2r)


def kernel(input_molecule_representations, graph_representations,
           graphs_requiring_node_choices, W1, b1, W2, b2):
    n_sel = graphs_requiring_node_choices.shape[0]
    idx = graphs_requiring_node_choices.astype(jnp.int32)
    idx_pad = jnp.concatenate(
        [idx, jnp.zeros((N_PAD - n_sel,), jnp.int32)])
    x1, x2 = _gather_rows(
        input_molecule_representations, graph_representations, idx_pad)
    w1a = W1[:D]
    w1b = W1[D:]
    return _mlp(x1, x2, w1a, w1b, W2,
                b1.reshape(1, HID), b2.reshape(1, OUT), n_sel)
